# Initial kernel scaffold; baseline (speedup 1.0000x reference)
import jax
import jax.numpy as jnp
from jax.experimental import pallas as pl


def kernel(x, W_base, W_gate, W_A, W_B):
    raise NotImplementedError("write your pallas kernel here")


# speedup vs baseline: 1.3513x; 1.3513x over previous
import jax
import jax.numpy as jnp
from jax.experimental import pallas as pl
from jax.experimental.pallas import tpu as pltpu
from functools import partial

N = 8192
D = 4096
OUT = 4096
NE = 28
R = 8
RMOE = NE * R
SCALING = 2.0

BN = 256


def _fused_kernel(x_ref, wg_ref, wa_ref, wb_ref, wbt_ref, out_ref):
    xb = x_ref[...].astype(jnp.bfloat16)
    logits = jnp.dot(xb, wg_ref[...], preferred_element_type=jnp.float32)
    ii = jax.lax.broadcasted_iota(jnp.int32, (BN, NE), 1)
    m1 = jnp.max(logits, axis=-1, keepdims=True)
    i1 = jnp.min(jnp.where(logits == m1, ii, NE), axis=-1, keepdims=True)
    l2 = jnp.where(ii == i1, -jnp.inf, logits)
    m2 = jnp.max(l2, axis=-1, keepdims=True)
    i2 = jnp.min(jnp.where(l2 == m2, ii, NE), axis=-1, keepdims=True)
    e = jnp.exp(m2 - m1)
    w1 = 1.0 / (1.0 + e)
    w2 = 1.0 - w1
    colmap = jax.lax.broadcasted_iota(jnp.int32, (BN, RMOE), 1) // R
    gex = jnp.where(colmap == i1, w1, 0.0) + jnp.where(colmap == i2, w2, 0.0)
    h = jnp.dot(xb, wa_ref[...], preferred_element_type=jnp.float32)
    hw = (h * gex).astype(jnp.bfloat16)
    acc = jnp.dot(xb, wb_ref[...], preferred_element_type=jnp.float32)
    acc = acc + jnp.dot(hw, wbt_ref[...], preferred_element_type=jnp.float32)
    out_ref[...] = acc


@jax.jit
def kernel(x, W_base, W_gate, W_A, W_B):
    wg = W_gate.T.astype(jnp.bfloat16)
    wa = W_A.T.astype(jnp.bfloat16)
    wb = W_base.T.astype(jnp.bfloat16)
    wbt = (SCALING * W_B).T.astype(jnp.bfloat16)

    grid = (N // BN,)
    return pl.pallas_call(
        _fused_kernel,
        grid=grid,
        in_specs=[
            pl.BlockSpec((BN, D), lambda i: (i, 0)),
            pl.BlockSpec((D, NE), lambda i: (0, 0)),
            pl.BlockSpec((D, RMOE), lambda i: (0, 0)),
            pl.BlockSpec((D, OUT), lambda i: (0, 0)),
            pl.BlockSpec((RMOE, OUT), lambda i: (0, 0)),
        ],
        out_specs=pl.BlockSpec((BN, OUT), lambda i: (i, 0)),
        out_shape=jax.ShapeDtypeStruct((N, OUT), jnp.float32),
        compiler_params=pltpu.CompilerParams(
            vmem_limit_bytes=110 * 1024 * 1024,
        ),
    )(x, wg, wa, wb, wbt)
